# baseline (speedup 1.0000x reference)
import functools

import jax
import jax.numpy as jnp
from jax import lax
from jax.experimental import pallas as pl
from jax.experimental.pallas import tpu as pltpu
from jax.experimental.pallas import tpu_sc as plsc

N_NODES = 10000
E_EDGES = 320000
IN_CH = 128
HID = 64
OUT = 64
K_HOPS = 10
ALPHA = 0.1

NC = 2
NS = 16
NW = NC * NS
LANES = 16

NPT = 640
N_PAD = NS * NPT
C_EDGE = 128
E_TOT = E_EDGES + N_NODES
N_CHUNK = -(-E_TOT // (NW * C_EDGE))
E_PAD = NW * N_CHUNK * C_EDGE
NBUF = 3
N_CHUNK_A = N_CHUNK + NBUF
NZC = NPT // C_EDGE


def _dense_tc_kernel(xc_ref, xn_ref, w1_ref, w2_ref, skip_ref, hid_ref):
    dn = (((1,), (1,)), ((), ()))
    w1 = w1_ref[...]
    w2 = w2_ref[...]
    hc = jnp.maximum(lax.dot_general(xc_ref[...], w1, dn,
                                     preferred_element_type=jnp.float32), 0.0)
    skip_ref[...] = lax.dot_general(hc, w2, dn,
                                    preferred_element_type=jnp.float32)
    hn = jnp.maximum(lax.dot_general(xn_ref[...], w1, dn,
                                     preferred_element_type=jnp.float32), 0.0)
    hid_ref[...] = lax.dot_general(hn, w2, dn,
                                   preferred_element_type=jnp.float32)


def _prep_tc_kernel(degp_ref, h0_ref, dinv_ref, g0_ref):
    deg = degp_ref[0] + degp_ref[1]
    d = deg[:, 0:1]
    dinv = jnp.where(d > 0.0, lax.rsqrt(jnp.maximum(d, 1e-12)), 0.0)
    dinv_b = jnp.broadcast_to(dinv, (N_PAD, OUT))
    dinv_ref[...] = dinv_b
    g0_ref[...] = dinv_b * h0_ref[...]


def _update_tc_kernel(p_ref, dinv_ref, h0_ref, g_ref):
    s = p_ref[0] + p_ref[1]
    dinv = dinv_ref[...]
    g_ref[...] = ((1.0 - ALPHA) * dinv * dinv * s
                  + ALPHA * dinv * h0_ref[...])


def _final_tc_kernel(p_ref, dinv_ref, h0_ref, skip_ref, out_ref):
    s = p_ref[0] + p_ref[1]
    out_ref[...] = ((1.0 - ALPHA) * dinv_ref[...] * s
                    + ALPHA * h0_ref[...] + skip_ref[...])


def _sc_deg_body(row_hbm, degp_hbm, row_v, ones_v, zsm, deg_sh):
    cid = lax.axis_index("c")
    sid = lax.axis_index("s")
    nbase = sid * NPT

    pltpu.sync_copy(row_hbm.at[cid, sid], row_v)

    ones16 = jnp.ones((LANES,), jnp.float32)
    zero16 = jnp.zeros((LANES,), jnp.float32)

    def fill_const(i, _):
        ones_v[i, :] = ones16
        zsm[i, :] = zero16
        return 0
    lax.fori_loop(0, C_EDGE, fill_const, 0)

    for z in range(NZC):
        pltpu.sync_copy(zsm, deg_sh.at[pl.ds(nbase + z * C_EDGE, C_EDGE)])
    plsc.subcore_barrier()

    def chunk(j, _):
        pltpu.sync_copy(ones_v, deg_sh.at[row_v.at[j]], add=True)
        return 0
    lax.fori_loop(0, N_CHUNK, chunk, 0)
    plsc.subcore_barrier()

    pltpu.sync_copy(deg_sh.at[pl.ds(nbase, NPT)],
                    degp_hbm.at[cid, pl.ds(nbase, NPT)])


def _sc_scatter_body(g_hbm, row_hbm, col_hbm, part_hbm,
                     row_v, col_v, gb0, gb1, gb2, zsm,
                     gs0, gs1, gs2, ss0, ss1, ss2, acc_sh):
    cid = lax.axis_index("c")
    sid = lax.axis_index("s")
    nbase = sid * NPT
    gbufs = (gb0, gb1, gb2)
    gsems = (gs0, gs1, gs2)
    ssems = (ss0, ss1, ss2)

    pltpu.sync_copy(row_hbm.at[cid, sid], row_v)
    pltpu.sync_copy(col_hbm.at[cid, sid], col_v)

    zero16 = jnp.zeros((LANES,), jnp.float32)

    def fill_zero(i, _):
        for k in range(OUT // LANES):
            zsm[i, pl.ds(k * LANES, LANES)] = zero16
        return 0
    lax.fori_loop(0, C_EDGE, fill_zero, 0)

    for z in range(NZC):
        pltpu.sync_copy(zsm, acc_sh.at[pl.ds(nbase + z * C_EDGE, C_EDGE)])
    plsc.subcore_barrier()

    for b in range(NBUF):
        pltpu.async_copy(g_hbm.at[col_v.at[b]], gbufs[b], gsems[b])

    def trio(jj, _):
        j0 = jj * NBUF
        for b in range(NBUF):
            pltpu.make_async_copy(g_hbm.at[col_v.at[0]], gbufs[b],
                                  gsems[b]).wait()
            pltpu.async_copy(gbufs[b], acc_sh.at[row_v.at[j0 + b]],
                             ssems[b], add=True)
        for b in range(NBUF):
            pltpu.make_async_copy(gbufs[b], acc_sh.at[row_v.at[0]],
                                  ssems[b]).wait()
            pltpu.async_copy(g_hbm.at[col_v.at[j0 + b + NBUF]], gbufs[b],
                             gsems[b])
        return 0
    lax.fori_loop(0, N_CHUNK // NBUF, trio, 0)

    for b in range(NBUF):
        pltpu.make_async_copy(g_hbm.at[col_v.at[0]], gbufs[b],
                              gsems[b]).wait()
    plsc.subcore_barrier()

    pltpu.sync_copy(acc_sh.at[pl.ds(nbase, NPT)],
                    part_hbm.at[cid, pl.ds(nbase, NPT)])


_MESH = plsc.VectorSubcoreMesh(core_axis_name="c", subcore_axis_name="s")
_SC_PARAMS = pltpu.CompilerParams(use_tc_tiling_on_sc=False)

_deg_call = pl.kernel(
    _sc_deg_body,
    out_type=jax.ShapeDtypeStruct((NC, N_PAD, LANES), jnp.float32),
    mesh=_MESH,
    compiler_params=_SC_PARAMS,
    scratch_types=[
        pltpu.VMEM((N_CHUNK_A, C_EDGE), jnp.int32),
        pltpu.VMEM((C_EDGE, LANES), jnp.float32),
        pltpu.VMEM((C_EDGE, LANES), jnp.float32),
        pltpu.VMEM_SHARED((N_PAD, LANES), jnp.float32),
    ],
)

_scatter_call = pl.kernel(
    _sc_scatter_body,
    out_type=jax.ShapeDtypeStruct((NC, N_PAD, OUT), jnp.float32),
    mesh=_MESH,
    compiler_params=_SC_PARAMS,
    scratch_types=(
        [pltpu.VMEM((N_CHUNK_A, C_EDGE), jnp.int32)] * 2
        + [pltpu.VMEM((C_EDGE, OUT), jnp.float32)] * 3
        + [pltpu.VMEM((C_EDGE, OUT), jnp.float32)]
        + [pltpu.SemaphoreType.DMA] * 6
        + [pltpu.VMEM_SHARED((N_PAD, OUT), jnp.float32)]
    ),
)


def kernel(x_clean, x_noised, edge_index, W1, W2):
    skip, hidden = pl.pallas_call(
        _dense_tc_kernel,
        out_shape=[
            jax.ShapeDtypeStruct((N_NODES, OUT), jnp.float32),
            jax.ShapeDtypeStruct((N_NODES, OUT), jnp.float32),
        ],
    )(x_clean, x_noised, W1, W2)

    loop = jnp.arange(N_NODES, dtype=jnp.int32)
    pad = jnp.full((E_PAD - E_TOT,), N_NODES, dtype=jnp.int32)
    tail = jnp.full((NC, NS, NBUF, C_EDGE), N_NODES, dtype=jnp.int32)
    row = jnp.concatenate([edge_index[0], loop, pad]).reshape(
        NC, NS, N_CHUNK, C_EDGE)
    col = jnp.concatenate([edge_index[1], loop, pad]).reshape(
        NC, NS, N_CHUNK, C_EDGE)
    row = jnp.concatenate([row, tail], axis=2)
    col = jnp.concatenate([col, tail], axis=2)

    pad_rows = jnp.zeros((N_PAD - N_NODES, OUT), jnp.float32)
    h0 = jnp.concatenate([hidden, pad_rows])
    skp = jnp.concatenate([skip, pad_rows])

    degp = _deg_call(row)
    dinv_b, g = pl.pallas_call(
        _prep_tc_kernel,
        out_shape=[
            jax.ShapeDtypeStruct((N_PAD, OUT), jnp.float32),
            jax.ShapeDtypeStruct((N_PAD, OUT), jnp.float32),
        ],
    )(degp, h0)

    upd = pl.pallas_call(
        _update_tc_kernel,
        out_shape=jax.ShapeDtypeStruct((N_PAD, OUT), jnp.float32),
    )
    for _ in range(K_HOPS - 1):
        part = _scatter_call(g, row, col)
        g = upd(part, dinv_b, h0)
    part = _scatter_call(g, row, col)

    out_pad = pl.pallas_call(
        _final_tc_kernel,
        out_shape=jax.ShapeDtypeStruct((N_PAD, OUT), jnp.float32),
    )(part, dinv_b, h0, skp)
    return out_pad[:N_NODES]


# speedup vs baseline: 3.4782x; 3.4782x over previous
import jax
import jax.numpy as jnp
from jax import lax
from jax.experimental import pallas as pl
from jax.experimental.pallas import tpu as pltpu
from jax.experimental.pallas import tpu_sc as plsc

N_NODES = 10000
E_EDGES = 320000
IN_CH = 128
HID = 64
OUT = 64
K_HOPS = 10
ALPHA = 0.1

NC = 2
NS = 16
LANES = 16
FH = OUT // 2

NPT = 640
N_PAD = NS * NPT
C_EDGE = 128
E_TOT = E_EDGES + N_NODES
N_CHUNK = -(-E_TOT // (NS * C_EDGE))
E_PAD = NS * N_CHUNK * C_EDGE
NBUF = 3
N_CHUNK_A = N_CHUNK + NBUF
NZC = NPT // C_EDGE


def _dense_tc_kernel(xc_ref, xn_ref, w1_ref, w2_ref, skip_ref, hid_ref):
    dn = (((1,), (1,)), ((), ()))
    w1 = w1_ref[...]
    w2 = w2_ref[...]
    hc = jnp.maximum(lax.dot_general(xc_ref[...], w1, dn,
                                     preferred_element_type=jnp.float32), 0.0)
    skip_ref[...] = lax.dot_general(hc, w2, dn,
                                    preferred_element_type=jnp.float32)
    hn = jnp.maximum(lax.dot_general(xn_ref[...], w1, dn,
                                     preferred_element_type=jnp.float32), 0.0)
    hid_ref[...] = lax.dot_general(hn, w2, dn,
                                   preferred_element_type=jnp.float32)


def _rsqrt_newton(d):
    y = lax.bitcast_convert_type(
        jnp.int32(0x5F3759DF) - lax.shift_right_logical(
            lax.bitcast_convert_type(d, jnp.int32), 1),
        jnp.float32)
    half_d = 0.5 * d
    for _ in range(4):
        y = y * (1.5 - half_d * y * y)
    return jnp.where(d > 0.0, y, 0.0)


def _sc_body(h0_hbm, skip_hbm, row_hbm, col_hbm,
             out_hbm,
             col_v, rx0, rx1, rx2, gb0, gb1, gb2, zsmall, accbuf,
             h0buf, dinv_v,
             gs0, gs1, gs2, ss0, ss1, ss2, rs0, rs1, rs2,
             acc_sh, g_sh):
    cid = lax.axis_index("c")
    sid = lax.axis_index("s")
    nbase = sid * NPT
    ridxs = (rx0, rx1, rx2)
    gbufs = (gb0, gb1, gb2)
    gsems = (gs0, gs1, gs2)
    ssems = (ss0, ss1, ss2)
    rsems = (rs0, rs1, rs2)

    pltpu.sync_copy(col_hbm.at[sid], col_v)

    ones16 = jnp.ones((LANES,), jnp.float32)
    zero16 = jnp.zeros((LANES,), jnp.float32)

    def fill_const(i, _):
        gb0[i, pl.ds(0, LANES)] = ones16
        gb0[i, pl.ds(LANES, LANES)] = ones16
        zsmall[i, pl.ds(0, LANES)] = zero16
        zsmall[i, pl.ds(LANES, LANES)] = zero16
        return 0
    lax.fori_loop(0, C_EDGE, fill_const, 0)

    for z in range(NZC):
        pltpu.sync_copy(zsmall, acc_sh.at[pl.ds(nbase + z * C_EDGE, C_EDGE)])
    plsc.subcore_barrier()

    for b in range(NBUF):
        pltpu.async_copy(row_hbm.at[sid, b], ridxs[b], rsems[b])

    def deg_trio(jj, _):
        j0 = jj * NBUF
        for b in range(NBUF):
            pltpu.make_async_copy(row_hbm.at[sid, 0], ridxs[b],
                                  rsems[b]).wait()
            pltpu.sync_copy(gb0, acc_sh.at[ridxs[b]], add=True)
            pltpu.async_copy(row_hbm.at[sid, j0 + b + NBUF], ridxs[b],
                             rsems[b])
        return 0
    lax.fori_loop(0, N_CHUNK // NBUF, deg_trio, 0)
    for b in range(NBUF):
        pltpu.make_async_copy(row_hbm.at[sid, 0], ridxs[b], rsems[b]).wait()
    plsc.subcore_barrier()

    pltpu.sync_copy(acc_sh.at[pl.ds(nbase, NPT)], accbuf)
    pltpu.sync_copy(h0_hbm.at[cid, pl.ds(nbase, NPT)], h0buf)

    def node_init(n, _):
        d = accbuf[n, pl.ds(0, LANES)]
        dinv = _rsqrt_newton(d)
        dinv_v[n, :] = dinv
        for k in range(FH // LANES):
            h = h0buf[n, pl.ds(k * LANES, LANES)]
            h0buf[n, pl.ds(k * LANES, LANES)] = dinv * h
        return 0
    lax.fori_loop(0, NPT, node_init, 0)

    pltpu.sync_copy(h0buf, g_sh.at[pl.ds(nbase, NPT)])

    for z in range(NZC):
        pltpu.sync_copy(zsmall, acc_sh.at[pl.ds(nbase + z * C_EDGE, C_EDGE)])
    plsc.subcore_barrier()

    def scatter_phase():
        for b in range(NBUF):
            pltpu.async_copy(row_hbm.at[sid, b], ridxs[b], rsems[b])
            pltpu.async_copy(g_sh.at[col_v.at[b]], gbufs[b], gsems[b])

        def trio(jj, _):
            j0 = jj * NBUF
            for b in range(NBUF):
                pltpu.make_async_copy(g_sh.at[col_v.at[0]], gbufs[b],
                                      gsems[b]).wait()
                pltpu.make_async_copy(row_hbm.at[sid, 0], ridxs[b],
                                      rsems[b]).wait()
                pltpu.async_copy(gbufs[b], acc_sh.at[ridxs[b]], ssems[b],
                                 add=True)
            for b in range(NBUF):
                pltpu.make_async_copy(gbufs[b], acc_sh.at[ridxs[b]],
                                      ssems[b]).wait()
                pltpu.async_copy(row_hbm.at[sid, j0 + b + NBUF], ridxs[b],
                                 rsems[b])
                pltpu.async_copy(g_sh.at[col_v.at[j0 + b + NBUF]], gbufs[b],
                                 gsems[b])
            return 0
        lax.fori_loop(0, N_CHUNK // NBUF, trio, 0)

        for b in range(NBUF):
            pltpu.make_async_copy(g_sh.at[col_v.at[0]], gbufs[b],
                                  gsems[b]).wait()
            pltpu.make_async_copy(row_hbm.at[sid, 0], ridxs[b],
                                  rsems[b]).wait()

    def hop(t, _):
        scatter_phase()
        plsc.subcore_barrier()

        pltpu.sync_copy(acc_sh.at[pl.ds(nbase, NPT)], accbuf)
        for z in range(NZC):
            pltpu.sync_copy(zsmall,
                            acc_sh.at[pl.ds(nbase + z * C_EDGE, C_EDGE)])

        @pl.when(t < K_HOPS - 1)
        def _():
            pltpu.sync_copy(h0_hbm.at[cid, pl.ds(nbase, NPT)], h0buf)

            def upd(n, _):
                dinv = dinv_v[n, :]
                c2 = (1.0 - ALPHA) * dinv * dinv
                adinv = ALPHA * dinv
                for k in range(FH // LANES):
                    s = accbuf[n, pl.ds(k * LANES, LANES)]
                    accbuf[n, pl.ds(k * LANES, LANES)] = (
                        c2 * s + adinv * h0buf[n, pl.ds(k * LANES, LANES)])
                return 0
            lax.fori_loop(0, NPT, upd, 0)
            pltpu.sync_copy(accbuf, g_sh.at[pl.ds(nbase, NPT)])

        @pl.when(t == K_HOPS - 1)
        def _():
            pltpu.sync_copy(h0_hbm.at[cid, pl.ds(nbase, NPT)], h0buf)

            def fin(n, _):
                dinv = dinv_v[n, :]
                for k in range(FH // LANES):
                    s = accbuf[n, pl.ds(k * LANES, LANES)]
                    accbuf[n, pl.ds(k * LANES, LANES)] = (
                        (1.0 - ALPHA) * dinv * s
                        + ALPHA * h0buf[n, pl.ds(k * LANES, LANES)])
                return 0
            lax.fori_loop(0, NPT, fin, 0)
            pltpu.sync_copy(skip_hbm.at[cid, pl.ds(nbase, NPT)], h0buf)

            def fin2(n, _):
                for k in range(FH // LANES):
                    accbuf[n, pl.ds(k * LANES, LANES)] = (
                        accbuf[n, pl.ds(k * LANES, LANES)]
                        + h0buf[n, pl.ds(k * LANES, LANES)])
                return 0
            lax.fori_loop(0, NPT, fin2, 0)
            pltpu.sync_copy(accbuf, out_hbm.at[cid, pl.ds(nbase, NPT)])

        plsc.subcore_barrier()
        return 0

    lax.fori_loop(0, K_HOPS, hop, 0)


def kernel(x_clean, x_noised, edge_index, W1, W2):
    skip, hidden = pl.pallas_call(
        _dense_tc_kernel,
        out_shape=[
            jax.ShapeDtypeStruct((N_NODES, OUT), jnp.float32),
            jax.ShapeDtypeStruct((N_NODES, OUT), jnp.float32),
        ],
    )(x_clean, x_noised, W1, W2)

    loop = jnp.arange(N_NODES, dtype=jnp.int32)
    pad = jnp.full((E_PAD - E_TOT,), N_NODES, dtype=jnp.int32)
    tail = jnp.full((NS, NBUF, C_EDGE), N_NODES, dtype=jnp.int32)
    row = jnp.concatenate([edge_index[0], loop, pad]).reshape(NS, N_CHUNK, C_EDGE)
    col = jnp.concatenate([edge_index[1], loop, pad]).reshape(NS, N_CHUNK, C_EDGE)
    row = jnp.concatenate([row, tail], axis=1)
    col = jnp.concatenate([col, tail], axis=1)

    pad_rows = jnp.zeros((N_PAD - N_NODES, OUT), jnp.float32)
    h0_pad = jnp.concatenate([hidden, pad_rows])
    skip_pad = jnp.concatenate([skip, pad_rows])
    h0_halves = h0_pad.reshape(N_PAD, NC, FH).transpose(1, 0, 2)
    skip_halves = skip_pad.reshape(N_PAD, NC, FH).transpose(1, 0, 2)

    mesh = plsc.VectorSubcoreMesh(core_axis_name="c", subcore_axis_name="s")

    sc_call = pl.kernel(
        _sc_body,
        out_type=jax.ShapeDtypeStruct((NC, N_PAD, FH), jnp.float32),
        mesh=mesh,
        compiler_params=pltpu.CompilerParams(use_tc_tiling_on_sc=False),
        scratch_types=(
            [pltpu.VMEM((N_CHUNK_A, C_EDGE), jnp.int32)]
            + [pltpu.VMEM((C_EDGE,), jnp.int32)] * 3
            + [pltpu.VMEM((C_EDGE, FH), jnp.float32)] * 3
            + [pltpu.VMEM((C_EDGE, FH), jnp.float32)]
            + [pltpu.VMEM((NPT, FH), jnp.float32)] * 2
            + [pltpu.VMEM((NPT, LANES), jnp.float32)]
            + [pltpu.SemaphoreType.DMA] * 9
            + [pltpu.VMEM_SHARED((N_PAD, FH), jnp.float32)] * 2
        ),
    )

    out_halves = sc_call(h0_halves, skip_halves, row, col)
    out_pad = out_halves.transpose(1, 0, 2).reshape(N_PAD, OUT)
    return out_pad[:N_NODES]
